# baseline (speedup 1.0000x reference)
import functools
import math

import jax
import jax.numpy as jnp
from jax import lax
from jax.experimental import pallas as pl
from jax.experimental.pallas import tpu as pltpu
from jax.experimental.pallas import tpu_sc as plsc

_TBLK = 8192
_TQRT = 2048


def _scale_table(embed_mat):
    V, D = embed_mat.shape
    scale = math.sqrt(D)
    tabT = embed_mat.T
    blk = _TBLK
    qrt = _TQRT
    grid = pl.cdiv(V, blk)
    vpad = grid * blk
    nq = 128 // D

    def body(t_ref, o_ref):
        v = t_ref[...]
        ss = jnp.sum(v * v, axis=0, keepdims=True)
        inv = scale * lax.rsqrt(jnp.maximum(ss, 1e-24))
        z = v * inv
        parts = [
            jnp.transpose(z[:, c * qrt:(c + 1) * qrt], (1, 0))
            for c in range(nq)
        ]
        o_ref[...] = jnp.concatenate(parts, axis=1)

    out = pl.pallas_call(
        body,
        grid=(grid,),
        in_specs=[pl.BlockSpec((D, blk), lambda i: (0, i))],
        out_specs=pl.BlockSpec((qrt, 128), lambda i: (i, 0)),
        out_shape=jax.ShapeDtypeStruct((grid * qrt, 128), jnp.float32),
    )(tabT)
    return out.reshape(vpad, D)


def _sc_gather(table, idx2d, H, B):
    Vp, D = table.shape
    GW = 128
    K = 4
    C = K * GW
    NBUF = 2
    N = H * B
    n_chunks = N // C
    bph = B // C

    info = plsc.get_sparse_core_info()
    NC, NS = info.num_cores, info.num_subcores
    NW = NC * NS
    per_w = n_chunks // NW
    assert n_chunks % (NW * NBUF) == 0 and bph & (bph - 1) == 0
    lb = bph.bit_length() - 1

    mesh = plsc.VectorSubcoreMesh(core_axis_name="c", subcore_axis_name="s")

    @functools.partial(
        pl.kernel,
        out_type=jax.ShapeDtypeStruct((H, D, B), jnp.float32),
        mesh=mesh,
        scratch_types=[
            pltpu.VMEM((K, GW), jnp.int32),
            pltpu.VMEM((NBUF * C, D), jnp.float32),
            pltpu.VMEM((NBUF * D, C), jnp.float32),
            pltpu.SemaphoreType.DMA,
            pltpu.SemaphoreType.DMA,
            pltpu.SemaphoreType.DMA,
        ],
        compiler_params=pltpu.CompilerParams(
            use_tc_tiling_on_sc=False, needs_layout_passes=False),
    )
    def gather_kernel(table_hbm, idx_hbm, out_hbm, idx_v, rows_v, col_v,
                      semg, semw0, semw1):
        wid = lax.axis_index("s") * NC + lax.axis_index("c")
        base_c = wid * per_w
        semw = (semw0, semw1)
        iota = lax.iota(jnp.int32, 16)

        def outer(i, carry):
            for b in range(NBUF):
                col_b = col_v.at[pl.ds(b * D, D)]

                @pl.when(i >= 1)
                def _drain_prev():
                    pltpu.make_async_copy(
                        col_b, out_hbm.at[0, :, pl.ds(0, C)], semw[b]).wait()

                cid = base_c + i * NBUF + b
                pltpu.sync_copy(idx_hbm.at[pl.ds(cid * K, K)], idx_v)
                for j in range(K):
                    for l in range(GW // 16):
                        w = idx_v[j, pl.ds(l * 16, 16)]
                        s = ((w & (-_TBLK)) | ((w & (_TQRT - 1)) << 2)
                             | ((w & (_TBLK - 1)) >> 11))
                        idx_v[j, pl.ds(l * 16, 16)] = s
                cps = [
                    pltpu.async_copy(
                        table_hbm.at[idx_v.at[j]],
                        rows_v.at[pl.ds(b * C + j * GW, GW)],
                        semg,
                    )
                    for j in range(K)
                ]
                for cp in cps:
                    cp.wait()

                rofs = b * C
                cofs = b * D

                def tbody(t, cr):
                    d = t >> ((C // 16).bit_length() - 1)
                    g = t & (C // 16 - 1)
                    rvec = iota + (rofs + g * 16)
                    v = plsc.load_gather(
                        rows_v, [rvec, jnp.full((16,), d, jnp.int32)])
                    scvec = iota + g * 16
                    plsc.store_scatter(
                        col_v, [jnp.full((16,), cofs + d, jnp.int32), scvec],
                        v)
                    return cr

                lax.fori_loop(0, (C // 16) * D, tbody, 0)

                h = cid >> lb
                b0 = (cid & (bph - 1)) * C
                pltpu.async_copy(col_b, out_hbm.at[h, :, pl.ds(b0, C)],
                                 semw[b])
            return carry

        lax.fori_loop(0, per_w // NBUF, outer, 0)
        for b in range(NBUF):
            pltpu.make_async_copy(
                col_v.at[pl.ds(b * D, D)], out_hbm.at[0, :, pl.ds(0, C)],
                semw[b]).wait()

    return gather_kernel(table, idx2d)


def kernel(embed_mat, x):
    B, H = x.shape
    _, D = embed_mat.shape
    table = _scale_table(embed_mat)
    idx2d = x.T.astype(jnp.int32).reshape(H * B // 128, 128)
    out3 = _sc_gather(table, idx2d, H, B)
    return jnp.transpose(out3, (2, 0, 1))


# speedup vs baseline: 1.2040x; 1.2040x over previous
import functools
import math

import jax
import jax.numpy as jnp
from jax import lax
from jax.experimental import pallas as pl
from jax.experimental.pallas import tpu as pltpu
from jax.experimental.pallas import tpu_sc as plsc

_TBLK = 8192
_TQRT = 2048


def _scale_table(embed_mat):
    V, D = embed_mat.shape
    scale = math.sqrt(D)
    tabT = embed_mat.T
    blk = _TBLK
    qrt = _TQRT
    grid = pl.cdiv(V, blk)
    vpad = grid * blk
    nq = 128 // D

    def body(t_ref, o_ref):
        v = t_ref[...]
        ss = jnp.sum(v * v, axis=0, keepdims=True)
        inv = scale * lax.rsqrt(jnp.maximum(ss, 1e-24))
        z = v * inv
        parts = [
            jnp.transpose(z[:, c * qrt:(c + 1) * qrt], (1, 0))
            for c in range(nq)
        ]
        o_ref[...] = jnp.concatenate(parts, axis=1)

    out = pl.pallas_call(
        body,
        grid=(grid,),
        in_specs=[pl.BlockSpec((D, blk), lambda i: (0, i))],
        out_specs=pl.BlockSpec((qrt, 128), lambda i: (i, 0)),
        out_shape=jax.ShapeDtypeStruct((grid * qrt, 128), jnp.float32),
    )(tabT)
    return out.reshape(vpad, D)


def _sc_gather(table, idx2d, H, B):
    Vp, D = table.shape
    GW = 128
    K = 4
    C = K * GW
    NBUF = 2
    N = H * B
    n_chunks = N // C
    bph = B // C

    info = plsc.get_sparse_core_info()
    NC, NS = info.num_cores, info.num_subcores
    NW = NC * NS
    per_w = n_chunks // NW
    assert n_chunks % (NW * NBUF) == 0 and bph & (bph - 1) == 0
    lb = bph.bit_length() - 1

    mesh = plsc.VectorSubcoreMesh(core_axis_name="c", subcore_axis_name="s")

    @functools.partial(
        pl.kernel,
        out_type=jax.ShapeDtypeStruct((H, D, B), jnp.float32),
        mesh=mesh,
        scratch_types=[
            pltpu.VMEM((K, GW), jnp.int32),
            pltpu.VMEM((NBUF * C, D), jnp.float32),
            pltpu.VMEM((NBUF * D, C), jnp.float32),
            pltpu.SemaphoreType.DMA,
            pltpu.SemaphoreType.DMA,
            pltpu.SemaphoreType.DMA,
        ],
        compiler_params=pltpu.CompilerParams(
            use_tc_tiling_on_sc=False, needs_layout_passes=False),
    )
    def gather_kernel(table_hbm, idx_hbm, out_hbm, idx_v, rows_v, col_v,
                      semg, semw0, semw1):
        wid = lax.axis_index("s") * NC + lax.axis_index("c")
        base_c = wid * per_w
        semw = (semw0, semw1)
        iota = lax.iota(jnp.int32, 16)

        def outer(i, carry):
            for b in range(NBUF):
                col_b = col_v.at[pl.ds(b * D, D)]

                @pl.when(i >= 1)
                def _drain_prev():
                    pltpu.make_async_copy(
                        col_b, out_hbm.at[0, :, pl.ds(0, C)], semw[b]).wait()

                cid = base_c + i * NBUF + b
                pltpu.sync_copy(idx_hbm.at[pl.ds(cid * K, K)], idx_v)
                for j in range(K):
                    for l in range(GW // 16):
                        w = idx_v[j, pl.ds(l * 16, 16)]
                        s = ((w & (-_TBLK)) | ((w & (_TQRT - 1)) << 2)
                             | ((w & (_TBLK - 1)) >> 11))
                        idx_v[j, pl.ds(l * 16, 16)] = s
                cps = [
                    pltpu.async_copy(
                        table_hbm.at[idx_v.at[j]],
                        rows_v.at[pl.ds(b * C + j * GW, GW)],
                        semg,
                    )
                    for j in range(K)
                ]
                for cp in cps:
                    cp.wait()

                rofs = b * C
                cofs = b * D
                for d in range(D):
                    cvec = jnp.full((16,), d, jnp.int32)
                    srvec = jnp.full((16,), cofs + d, jnp.int32)

                    @plsc.parallel_loop(0, C // 16, unroll=4)
                    def _t(g, _d=d, _cv=cvec, _sv=srvec):
                        base = iota + g * 16
                        v = plsc.load_gather(rows_v, [base + rofs, _cv])
                        plsc.store_scatter(col_v, [_sv, base], v)

                h = cid >> lb
                b0 = (cid & (bph - 1)) * C
                pltpu.async_copy(col_b, out_hbm.at[h, :, pl.ds(b0, C)],
                                 semw[b])
            return carry

        lax.fori_loop(0, per_w // NBUF, outer, 0)
        for b in range(NBUF):
            pltpu.make_async_copy(
                col_v.at[pl.ds(b * D, D)], out_hbm.at[0, :, pl.ds(0, C)],
                semw[b]).wait()

    return gather_kernel(table, idx2d)


def kernel(embed_mat, x):
    B, H = x.shape
    _, D = embed_mat.shape
    table = _scale_table(embed_mat)
    idx2d = x.T.astype(jnp.int32).reshape(H * B // 128, 128)
    out3 = _sc_gather(table, idx2d, H, B)
    return jnp.transpose(out3, (2, 0, 1))


# speedup vs baseline: 2.3594x; 1.9596x over previous
import functools
import math

import jax
import jax.numpy as jnp
from jax import lax
from jax.experimental import pallas as pl
from jax.experimental.pallas import tpu as pltpu
from jax.experimental.pallas import tpu_sc as plsc

_TBLK = 8192
_TQRT = 2048


def _scale_table(embed_mat):
    V, D = embed_mat.shape
    scale = math.sqrt(D)
    tabT = embed_mat.T
    blk = _TBLK
    qrt = _TQRT
    grid = pl.cdiv(V, blk)
    vpad = grid * blk
    nq = 128 // D

    def body(t_ref, o_ref):
        v = t_ref[...]
        ss = jnp.sum(v * v, axis=0, keepdims=True)
        inv = scale * lax.rsqrt(jnp.maximum(ss, 1e-24))
        z = v * inv
        parts = [
            jnp.transpose(z[:, c * qrt:(c + 1) * qrt], (1, 0))
            for c in range(nq)
        ]
        o_ref[...] = jnp.concatenate(parts, axis=1)

    out = pl.pallas_call(
        body,
        grid=(grid,),
        in_specs=[pl.BlockSpec((D, blk), lambda i: (0, i))],
        out_specs=pl.BlockSpec((qrt, 128), lambda i: (i, 0)),
        out_shape=jax.ShapeDtypeStruct((grid * qrt, 128), jnp.float32),
    )(tabT)
    return out.reshape(vpad, D)


def _sc_gather(table, idx2d, H, B):
    Vp, D = table.shape
    GW = 128
    K = 4
    C = K * GW
    NBUF = 2
    N = H * B
    n_chunks = N // C
    bph = B // C

    info = plsc.get_sparse_core_info()
    NC, NS = info.num_cores, info.num_subcores
    NW = NC * NS
    per_w = n_chunks // NW
    assert n_chunks % (NW * NBUF) == 0 and bph & (bph - 1) == 0
    lb = bph.bit_length() - 1

    mesh = plsc.VectorSubcoreMesh(core_axis_name="c", subcore_axis_name="s")

    @functools.partial(
        pl.kernel,
        out_type=jax.ShapeDtypeStruct((H, D, B), jnp.float32),
        mesh=mesh,
        scratch_types=[
            pltpu.VMEM((K, GW), jnp.int32),
            pltpu.VMEM((NBUF * C, D), jnp.float32),
            pltpu.VMEM((NBUF * D, C), jnp.float32),
            pltpu.SemaphoreType.DMA,
            pltpu.SemaphoreType.DMA,
            pltpu.SemaphoreType.DMA,
        ],
        compiler_params=pltpu.CompilerParams(
            use_tc_tiling_on_sc=False, needs_layout_passes=False),
    )
    def gather_kernel(table_hbm, idx_hbm, out_hbm, idx_v, rows_v, col_v,
                      semg, semw0, semw1):
        wid = lax.axis_index("s") * NC + lax.axis_index("c")
        base_c = wid * per_w
        semw = (semw0, semw1)
        iota = lax.iota(jnp.int32, 16)

        def outer(i, carry):
            for b in range(NBUF):
                col_b = col_v.at[pl.ds(b * D, D)]

                @pl.when(i >= 1)
                def _drain_prev():
                    pltpu.make_async_copy(
                        col_b, out_hbm.at[0, :, pl.ds(0, C)], semw[b]).wait()

                cid = base_c + i * NBUF + b
                pltpu.sync_copy(idx_hbm.at[pl.ds(cid * K, K)], idx_v)
                for j in range(K):
                    for l in range(GW // 16):
                        w = idx_v[j, pl.ds(l * 16, 16)]
                        s = ((w & (-_TBLK)) | ((w & (_TQRT - 1)) << 2)
                             | ((w & (_TBLK - 1)) >> 11))
                        idx_v[j, pl.ds(l * 16, 16)] = s
                cps = [
                    pltpu.async_copy(
                        table_hbm.at[idx_v.at[j]],
                        rows_v.at[pl.ds(b * C + j * GW, GW)],
                        semg,
                    )
                    for j in range(K)
                ]
                for cp in cps:
                    cp.wait()

                rofs = b * C
                cofs = b * D
                for d0 in range(D):
                    cvec = (iota + d0) & (D - 1)
                    srvec = cvec + cofs

                    @plsc.parallel_loop(0, C // 16, unroll=4)
                    def _t(g, _cv=cvec, _sv=srvec):
                        base = iota + g * 16
                        v = plsc.load_gather(rows_v, [base + rofs, _cv])
                        plsc.store_scatter(col_v, [_sv, base], v)

                h = cid >> lb
                b0 = (cid & (bph - 1)) * C
                pltpu.async_copy(col_b, out_hbm.at[h, :, pl.ds(b0, C)],
                                 semw[b])
            return carry

        lax.fori_loop(0, per_w // NBUF, outer, 0)
        for b in range(NBUF):
            pltpu.make_async_copy(
                col_v.at[pl.ds(b * D, D)], out_hbm.at[0, :, pl.ds(0, C)],
                semw[b]).wait()

    return gather_kernel(table, idx2d)


def kernel(embed_mat, x):
    B, H = x.shape
    _, D = embed_mat.shape
    table = _scale_table(embed_mat)
    idx2d = x.T.astype(jnp.int32).reshape(H * B // 128, 128)
    out3 = _sc_gather(table, idx2d, H, B)
    return jnp.transpose(out3, (2, 0, 1))


# speedup vs baseline: 2.7352x; 1.1593x over previous
import functools
import math

import jax
import jax.numpy as jnp
from jax import lax
from jax.experimental import pallas as pl
from jax.experimental.pallas import tpu as pltpu
from jax.experimental.pallas import tpu_sc as plsc

_TBLK = 8192
_TQRT = 2048


def _scale_table(embed_mat):
    V, D = embed_mat.shape
    scale = math.sqrt(D)
    tabT = embed_mat.T
    blk = _TBLK
    qrt = _TQRT
    grid = pl.cdiv(V, blk)
    vpad = grid * blk
    nq = 128 // D

    def body(t_ref, o_ref):
        v = t_ref[...]
        ss = jnp.sum(v * v, axis=0, keepdims=True)
        inv = scale * lax.rsqrt(jnp.maximum(ss, 1e-24))
        z = v * inv
        parts = [
            jnp.transpose(z[:, c * qrt:(c + 1) * qrt], (1, 0))
            for c in range(nq)
        ]
        o_ref[...] = jnp.concatenate(parts, axis=1)

    out = pl.pallas_call(
        body,
        grid=(grid,),
        in_specs=[pl.BlockSpec((D, blk), lambda i: (0, i))],
        out_specs=pl.BlockSpec((qrt, 128), lambda i: (i, 0)),
        out_shape=jax.ShapeDtypeStruct((grid * qrt, 128), jnp.float32),
    )(tabT)
    return out.reshape(vpad, D)


def _sc_gather(table, idx2d, H, B):
    Vp, D = table.shape
    GW = 128
    K = 4
    C = K * GW
    NBUF = 2
    N = H * B
    n_chunks = N // C
    bph = B // C

    info = plsc.get_sparse_core_info()
    NC, NS = info.num_cores, info.num_subcores
    NW = NC * NS
    per_w = n_chunks // NW
    assert n_chunks % (NW * NBUF) == 0 and bph & (bph - 1) == 0
    lb = bph.bit_length() - 1

    mesh = plsc.VectorSubcoreMesh(core_axis_name="c", subcore_axis_name="s")

    @functools.partial(
        pl.kernel,
        out_type=jax.ShapeDtypeStruct((H, D, B), jnp.float32),
        mesh=mesh,
        scratch_types=[
            pltpu.VMEM((NBUF * K, GW), jnp.int32),
            pltpu.VMEM((NBUF * C, D), jnp.float32),
            pltpu.VMEM((NBUF * D, C), jnp.float32),
            pltpu.SemaphoreType.DMA,
            pltpu.SemaphoreType.DMA,
            pltpu.SemaphoreType.DMA,
            pltpu.SemaphoreType.DMA,
        ],
        compiler_params=pltpu.CompilerParams(
            use_tc_tiling_on_sc=False, needs_layout_passes=False),
    )
    def gather_kernel(table_hbm, idx_hbm, out_hbm, idx_v, rows_v, col_v,
                      semg0, semg1, semw0, semw1):
        wid = lax.axis_index("s") * NC + lax.axis_index("c")
        base_c = wid * per_w
        semg = (semg0, semg1)
        semw = (semw0, semw1)
        iota = lax.iota(jnp.int32, 16)

        def load_and_fire(c, slot):
            pltpu.sync_copy(idx_hbm.at[pl.ds(c * K, K)],
                            idx_v.at[pl.ds(slot * K, K)])
            for j in range(K):
                row = slot * K + j
                for l in range(GW // 16):
                    w = idx_v[row, pl.ds(l * 16, 16)]
                    s = ((w & (-_TBLK)) | ((w & (_TQRT - 1)) << 2)
                         | ((w & (_TBLK - 1)) >> 11))
                    idx_v[row, pl.ds(l * 16, 16)] = s
            for j in range(K):
                pltpu.async_copy(
                    table_hbm.at[idx_v.at[slot * K + j]],
                    rows_v.at[pl.ds(slot * C + j * GW, GW)],
                    semg[slot])

        load_and_fire(base_c, 0)

        def outer(io, carry):
            for b in range(NBUF):
                i = io * NBUF + b
                c = base_c + i
                nb = 1 - b

                @pl.when(i < per_w - 1)
                def _prefetch():
                    load_and_fire(c + 1, nb)

                for j in range(K):
                    pltpu.make_async_copy(
                        table_hbm.at[pl.ds(0, GW)],
                        rows_v.at[pl.ds(b * C + j * GW, GW)],
                        semg[b]).wait()

                @pl.when(io >= 1)
                def _free_col():
                    pltpu.make_async_copy(
                        col_v.at[pl.ds(b * D, D)],
                        out_hbm.at[0, :, pl.ds(0, C)], semw[b]).wait()

                rofs = b * C
                cofs = b * D
                for d0 in range(D):
                    cvec = (iota + d0) & (D - 1)
                    srvec = cvec + cofs

                    @plsc.parallel_loop(0, C // 16, unroll=4)
                    def _t(g, _cv=cvec, _sv=srvec):
                        base = iota + g * 16
                        v = plsc.load_gather(rows_v, [base + rofs, _cv])
                        plsc.store_scatter(col_v, [_sv, base], v)

                h = c >> lb
                b0 = (c & (bph - 1)) * C
                pltpu.async_copy(col_v.at[pl.ds(b * D, D)],
                                 out_hbm.at[h, :, pl.ds(b0, C)], semw[b])
            return carry

        lax.fori_loop(0, per_w // NBUF, outer, 0)
        for b in range(NBUF):
            pltpu.make_async_copy(
                col_v.at[pl.ds(b * D, D)], out_hbm.at[0, :, pl.ds(0, C)],
                semw[b]).wait()

    return gather_kernel(table, idx2d)


def kernel(embed_mat, x):
    B, H = x.shape
    _, D = embed_mat.shape
    table = _scale_table(embed_mat)
    idx2d = x.T.astype(jnp.int32).reshape(H * B // 128, 128)
    out3 = _sc_gather(table, idx2d, H, B)
    return jnp.transpose(out3, (2, 0, 1))


# speedup vs baseline: 4.0227x; 1.4707x over previous
import functools
import math

import jax
import jax.numpy as jnp
from jax import lax
from jax.experimental import pallas as pl
from jax.experimental.pallas import tpu as pltpu
from jax.experimental.pallas import tpu_sc as plsc

_TBLK = 8192
_TQRT = 2048


def _scale_table(embed_mat):
    V, D = embed_mat.shape
    scale = math.sqrt(D)
    tabT = embed_mat.T
    blk = _TBLK
    qrt = _TQRT
    grid = pl.cdiv(V, blk)
    vpad = grid * blk
    nq = 128 // D

    def body(t_ref, o_ref):
        v = t_ref[...]
        ss = jnp.sum(v * v, axis=0, keepdims=True)
        inv = scale * lax.rsqrt(jnp.maximum(ss, 1e-24))
        z = v * inv
        parts = [
            jnp.transpose(z[:, c * qrt:(c + 1) * qrt], (1, 0))
            for c in range(nq)
        ]
        o_ref[...] = jnp.concatenate(parts, axis=1)

    out = pl.pallas_call(
        body,
        grid=(grid,),
        in_specs=[pl.BlockSpec((D, blk), lambda i: (0, i))],
        out_specs=pl.BlockSpec((qrt, 128), lambda i: (i, 0)),
        out_shape=jax.ShapeDtypeStruct((grid * qrt, 128), jnp.float32),
    )(tabT)
    return out.reshape(vpad, D)


def _sc_gather(table, idx2d, H, B):
    Vp, D = table.shape
    GW = 128
    K = 4
    C = K * GW
    NBUF = 2
    N = H * B
    n_chunks = N // C
    bph = B // C

    info = plsc.get_sparse_core_info()
    NC, NS = info.num_cores, info.num_subcores
    NW = NC * NS
    per_w = n_chunks // NW
    assert n_chunks % (NW * NBUF) == 0 and bph & (bph - 1) == 0
    lb = bph.bit_length() - 1

    mesh = plsc.VectorSubcoreMesh(core_axis_name="c", subcore_axis_name="s")

    @functools.partial(
        pl.kernel,
        out_type=jax.ShapeDtypeStruct((H, D // 8, B // 128, 8, 128),
                                      jnp.float32),
        mesh=mesh,
        scratch_types=[
            pltpu.VMEM((NBUF * K, GW), jnp.int32),
            pltpu.VMEM((NBUF * C, D), jnp.float32),
            pltpu.VMEM((NBUF * (D // 8) * (C // 128), 8, 128), jnp.float32),
            pltpu.SemaphoreType.DMA,
            pltpu.SemaphoreType.DMA,
            pltpu.SemaphoreType.DMA,
            pltpu.SemaphoreType.DMA,
        ],
        compiler_params=pltpu.CompilerParams(
            use_tc_tiling_on_sc=False, needs_layout_passes=False),
    )
    def gather_kernel(table_hbm, idx_hbm, out_hbm, idx_v, rows_v, col_v,
                      semg0, semg1, semw0, semw1):
        wid = lax.axis_index("s") * NC + lax.axis_index("c")
        base_c = wid * per_w
        semg = (semg0, semg1)
        semw = (semw0, semw1)
        iota = lax.iota(jnp.int32, 16)

        def load_and_fire(c, slot):
            pltpu.sync_copy(idx_hbm.at[pl.ds(c * K, K)],
                            idx_v.at[pl.ds(slot * K, K)])
            for j in range(K):
                row = slot * K + j
                for l in range(GW // 16):
                    w = idx_v[row, pl.ds(l * 16, 16)]
                    s = ((w & (-_TBLK)) | ((w & (_TQRT - 1)) << 2)
                         | ((w & (_TBLK - 1)) >> 11))
                    idx_v[row, pl.ds(l * 16, 16)] = s
            for j in range(K):
                pltpu.async_copy(
                    table_hbm.at[idx_v.at[slot * K + j]],
                    rows_v.at[pl.ds(slot * C + j * GW, GW)],
                    semg[slot])

        load_and_fire(base_c, 0)

        def outer(io, carry):
            for b in range(NBUF):
                i = io * NBUF + b
                c = base_c + i
                nb = 1 - b

                @pl.when(i < per_w - 1)
                def _prefetch():
                    load_and_fire(c + 1, nb)

                for j in range(K):
                    pltpu.make_async_copy(
                        table_hbm.at[pl.ds(0, GW)],
                        rows_v.at[pl.ds(b * C + j * GW, GW)],
                        semg[b]).wait()

                nt = (D // 8) * (C // 128)

                @pl.when(io >= 1)
                def _free_col():
                    for dt in range(D // 8):
                        pltpu.make_async_copy(
                            col_v.at[pl.ds(b * nt + dt * (C // 128),
                                           C // 128)],
                            out_hbm.at[0, 0, pl.ds(0, C // 128)],
                            semw[b]).wait()

                rofs = b * C
                for d0 in range(D):
                    cvec = (iota + d0) & (D - 1)
                    dtv = (cvec >> 3) << 2
                    divec = cvec & 7

                    @plsc.parallel_loop(0, C // 16, unroll=4)
                    def _t(g, _cv=cvec, _dtv=dtv, _div=divec):
                        base = iota + g * 16
                        v = plsc.load_gather(rows_v, [base + rofs, _cv])
                        rowv = _dtv + (b * nt + (g >> 3))
                        biv = iota + (g & 7) * 16
                        plsc.store_scatter(col_v, [rowv, _div, biv], v)

                h = c >> lb
                bt0 = (c & (bph - 1)) * (C // 128)
                for dt in range(D // 8):
                    pltpu.async_copy(
                        col_v.at[pl.ds(b * nt + dt * (C // 128), C // 128)],
                        out_hbm.at[h, dt, pl.ds(bt0, C // 128)], semw[b])
            return carry

        lax.fori_loop(0, per_w // NBUF, outer, 0)
        for b in range(NBUF):
            for dt in range(D // 8):
                pltpu.make_async_copy(
                    col_v.at[pl.ds(b * ((D // 8) * (C // 128))
                                   + dt * (C // 128), C // 128)],
                    out_hbm.at[0, 0, pl.ds(0, C // 128)], semw[b]).wait()

    return gather_kernel(table, idx2d)


def kernel(embed_mat, x):
    B, H = x.shape
    _, D = embed_mat.shape
    table = _scale_table(embed_mat)
    idx2d = x.T.astype(jnp.int32).reshape(H * B // 128, 128)
    out5 = _sc_gather(table, idx2d, H, B)
    return jnp.transpose(out5, (2, 4, 0, 1, 3)).reshape(B, H, D)
